# Initial kernel scaffold; baseline (speedup 1.0000x reference)
import jax
import jax.numpy as jnp
from jax.experimental import pallas as pl


def kernel(inputs, edge_index, edge_weight, W, b):
    raise NotImplementedError("write your pallas kernel here")


# speedup vs baseline: 4.1883x; 4.1883x over previous
import functools

import jax
import jax.numpy as jnp
from jax import lax
from jax.experimental import pallas as pl
from jax.experimental.pallas import tpu as pltpu
from jax.experimental.pallas import tpu_sc as plsc

NC = 2
NS = 16
NW = NC * NS
EC = 128
LANES = 16


def _mm_tanh_body(x_ref, w_ref, o_ref):
    o_ref[...] = jnp.tanh(
        lax.dot_general(x_ref[...], w_ref[...], (((1,), (0,)), ((), ())),
                        precision=lax.Precision.HIGHEST,
                        preferred_element_type=jnp.float32))


def _combine_body(a_ref, b_ref, o_ref):
    o_ref[...] = a_ref[...] + b_ref[...]


def _make_sc_agg(N, N_pad, D, C):
    mesh = plsc.VectorSubcoreMesh(core_axis_name="c", subcore_axis_name="s",
                                  num_cores=NC, num_subcores=NS)
    rows_per_tile = N_pad // NS

    @functools.partial(
        pl.kernel,
        out_type=(jax.ShapeDtypeStruct((N_pad, D), jnp.float32),
                  jax.ShapeDtypeStruct((N_pad, D), jnp.float32)),
        mesh=mesh,
        compiler_params=pltpu.CompilerParams(needs_layout_passes=False),
        scratch_types=[
            pltpu.VMEM((C, EC), jnp.int32),
            pltpu.VMEM((C, EC), jnp.int32),
            pltpu.VMEM((C, EC), jnp.float32),
            pltpu.VMEM((EC, D), jnp.float32),
            pltpu.VMEM_SHARED((N_pad, D), jnp.float32),
            pltpu.SemaphoreType.DMA,
        ],
    )
    def sc_agg(h_hbm, src_hbm, dst_hbm, w_hbm, p0_hbm, p1_hbm,
               src_v, dst_v, w_v, rows_v, acc_sh, sem):
        cid = lax.axis_index("c")
        sid = lax.axis_index("s")
        wid = sid * NC + cid

        def _zrow(r, carry):
            for j in range(D // LANES):
                rows_v[r, pl.ds(j * LANES, LANES)] = jnp.zeros((LANES,),
                                                               jnp.float32)
            return carry
        lax.fori_loop(0, EC, _zrow, 0)
        base = sid * rows_per_tile
        for k in range(rows_per_tile // EC):
            pltpu.sync_copy(rows_v, acc_sh.at[pl.ds(base + k * EC, EC)])

        pltpu.sync_copy(src_hbm.at[wid], src_v)
        pltpu.sync_copy(dst_hbm.at[wid], dst_v)
        pltpu.sync_copy(w_hbm.at[wid], w_v)
        plsc.subcore_barrier()

        def _chunk(c, carry):
            pltpu.async_copy(h_hbm.at[src_v.at[c]], rows_v, sem).wait()
            cc = jnp.full((LANES,), c, jnp.int32)

            def _edge(e, ecarry):
                ws = plsc.load_gather(
                    w_v, [cc, jnp.full((LANES,), e, jnp.int32)])
                for j in range(D // LANES):
                    sl = pl.ds(j * LANES, LANES)
                    rows_v[e, sl] = rows_v[e, sl] * ws
                return ecarry
            lax.fori_loop(0, EC, _edge, 0)
            pltpu.sync_copy(rows_v, acc_sh.at[dst_v.at[c]], add=True)
            return carry
        lax.fori_loop(0, C, _chunk, 0)
        plsc.subcore_barrier()

        @pl.when(cid == 0)
        def _():
            pltpu.sync_copy(acc_sh.at[pl.ds(base, rows_per_tile)],
                            p0_hbm.at[pl.ds(base, rows_per_tile)])

        @pl.when(cid == 1)
        def _():
            pltpu.sync_copy(acc_sh.at[pl.ds(base, rows_per_tile)],
                            p1_hbm.at[pl.ds(base, rows_per_tile)])

    return sc_agg


def kernel(inputs, edge_index, edge_weight, W, b):
    N, D = inputs.shape
    E = edge_weight.shape[0]

    BM = 2000
    h = pl.pallas_call(
        _mm_tanh_body,
        grid=(N // BM,),
        in_specs=[pl.BlockSpec((BM, D), lambda i: (i, 0)),
                  pl.BlockSpec((D, D), lambda i: (0, 0))],
        out_specs=pl.BlockSpec((BM, D), lambda i: (i, 0)),
        out_shape=jax.ShapeDtypeStruct((N, D), jnp.float32),
    )(inputs, W)

    per = NW * EC
    C = (E + per - 1) // per
    E_pad = C * per
    pad = E_pad - E
    src = jnp.concatenate(
        [edge_index[0], jnp.zeros((pad,), jnp.int32)]).reshape(NW, C, EC)
    dst = jnp.concatenate(
        [edge_index[1], jnp.zeros((pad,), jnp.int32)]).reshape(NW, C, EC)
    wts = jnp.concatenate(
        [edge_weight, jnp.zeros((pad,), jnp.float32)]).reshape(NW, C, EC)

    stripe = NS * EC
    N_pad = ((N + stripe - 1) // stripe) * stripe

    p0, p1 = _make_sc_agg(N, N_pad, D, C)(h, src, dst, wts)

    out = pl.pallas_call(
        _combine_body,
        grid=(N // BM,),
        in_specs=[pl.BlockSpec((BM, D), lambda i: (i, 0)),
                  pl.BlockSpec((BM, D), lambda i: (i, 0))],
        out_specs=pl.BlockSpec((BM, D), lambda i: (i, 0)),
        out_shape=jax.ShapeDtypeStruct((N, D), jnp.float32),
    )(p0, p1)
    return out


# speedup vs baseline: 4.8560x; 1.1594x over previous
import functools

import jax
import jax.numpy as jnp
from jax import lax
from jax.experimental import pallas as pl
from jax.experimental.pallas import tpu as pltpu
from jax.experimental.pallas import tpu_sc as plsc

NC = 2
NS = 16
NW = NC * NS
EC = 128
GC = 64
LANES = 16


def _mm_tanh_body(x_ref, w_ref, o_ref):
    o_ref[...] = jnp.tanh(
        lax.dot_general(x_ref[...], w_ref[...], (((1,), (0,)), ((), ())),
                        precision=lax.Precision.HIGHEST,
                        preferred_element_type=jnp.float32))


def _combine_body(a_ref, b_ref, o_ref):
    o_ref[...] = a_ref[...] + b_ref[...]


def _make_sc_agg(N, N_pad, D, C):
    mesh = plsc.VectorSubcoreMesh(core_axis_name="c", subcore_axis_name="s",
                                  num_cores=NC, num_subcores=NS)
    rows_per_tile = N_pad // NS

    @functools.partial(
        pl.kernel,
        out_type=(jax.ShapeDtypeStruct((N_pad, D), jnp.float32),
                  jax.ShapeDtypeStruct((N_pad, D), jnp.float32)),
        mesh=mesh,
        compiler_params=pltpu.CompilerParams(needs_layout_passes=False),
        scratch_types=[
            pltpu.VMEM((C, EC), jnp.int32),
            pltpu.VMEM((C, EC), jnp.int32),
            pltpu.VMEM((C, EC), jnp.float32),
            pltpu.VMEM((GC, D), jnp.float32),
            pltpu.VMEM((GC, D), jnp.float32),
            pltpu.VMEM_SHARED((N_pad, D), jnp.float32),
            pltpu.SemaphoreType.DMA,
            pltpu.SemaphoreType.DMA,
            pltpu.SemaphoreType.DMA,
            pltpu.SemaphoreType.DMA,
        ],
    )
    def sc_agg(h_hbm, src_hbm, dst_hbm, w_hbm, p0_hbm, p1_hbm,
               src_v, dst_v, w_v, rows_a, rows_b, acc_sh,
               gs_a, gs_b, ss_a, ss_b):
        cid = lax.axis_index("c")
        sid = lax.axis_index("s")
        wid = sid * NC + cid

        def _zrow(r, carry):
            for j in range(D // LANES):
                rows_a[r, pl.ds(j * LANES, LANES)] = jnp.zeros((LANES,),
                                                               jnp.float32)
            return carry
        lax.fori_loop(0, GC, _zrow, 0)
        base = sid * rows_per_tile
        for k in range(rows_per_tile // GC):
            pltpu.sync_copy(rows_a, acc_sh.at[pl.ds(base + k * GC, GC)])

        pltpu.sync_copy(src_hbm.at[wid], src_v)
        pltpu.sync_copy(dst_hbm.at[wid], dst_v)
        pltpu.sync_copy(w_hbm.at[wid], w_v)
        plsc.subcore_barrier()

        def _scale(rows_ref, r, h):
            rr = jnp.full((LANES,), r, jnp.int32)

            def _edge(e, ecarry):
                ws = plsc.load_gather(
                    w_v, [rr, jnp.full((LANES,), h * GC, jnp.int32) + e])
                for j in range(D // LANES):
                    sl = pl.ds(j * LANES, LANES)
                    rows_ref[e, sl] = rows_ref[e, sl] * ws
                return ecarry
            lax.fori_loop(0, GC, _edge, 0)

        def _gather(r, h, rows_ref, sem):
            pltpu.async_copy(
                h_hbm.at[src_v.at[r, pl.ds(h * GC, GC)]], rows_ref, sem)

        def _gwait(rows_ref, sem):
            pltpu.make_async_copy(h_hbm.at[src_v.at[0, pl.ds(0, GC)]],
                                  rows_ref, sem).wait()

        zidx = jnp.zeros((LANES,), jnp.int32)

        def _scatter(rows_ref, r, h, sem):
            for k in range(GC // LANES):
                dv = dst_v[r, pl.ds(h * GC + k * LANES, LANES)]
                pltpu.async_copy(rows_ref.at[pl.ds(k * LANES, LANES)],
                                 acc_sh.at[dv], sem, add=True)

        def _sdrain(rows_ref, sem):
            for k in range(GC // LANES):
                pltpu.make_async_copy(rows_ref.at[pl.ds(k * LANES, LANES)],
                                      acc_sh.at[zidx], sem).wait()

        _gather(0, 0, rows_a, gs_a)
        _gather(0, 1, rows_b, gs_b)

        def _row(r, carry):
            rn = jnp.minimum(r + 1, C - 1)
            _gwait(rows_a, gs_a)
            _scale(rows_a, r, 0)
            _scatter(rows_a, r, 0, ss_a)
            _gwait(rows_b, gs_b)
            _scale(rows_b, r, 1)
            _scatter(rows_b, r, 1, ss_b)
            _sdrain(rows_a, ss_a)
            _gather(rn, 0, rows_a, gs_a)
            _sdrain(rows_b, ss_b)
            _gather(rn, 1, rows_b, gs_b)
            return carry
        lax.fori_loop(0, C, _row, 0)
        _gwait(rows_a, gs_a)
        _gwait(rows_b, gs_b)
        plsc.subcore_barrier()

        @pl.when(cid == 0)
        def _():
            pltpu.sync_copy(acc_sh.at[pl.ds(base, rows_per_tile)],
                            p0_hbm.at[pl.ds(base, rows_per_tile)])

        @pl.when(cid == 1)
        def _():
            pltpu.sync_copy(acc_sh.at[pl.ds(base, rows_per_tile)],
                            p1_hbm.at[pl.ds(base, rows_per_tile)])

    return sc_agg


def kernel(inputs, edge_index, edge_weight, W, b):
    N, D = inputs.shape
    E = edge_weight.shape[0]

    BM = 2000
    h = pl.pallas_call(
        _mm_tanh_body,
        grid=(N // BM,),
        in_specs=[pl.BlockSpec((BM, D), lambda i: (i, 0)),
                  pl.BlockSpec((D, D), lambda i: (0, 0))],
        out_specs=pl.BlockSpec((BM, D), lambda i: (i, 0)),
        out_shape=jax.ShapeDtypeStruct((N, D), jnp.float32),
    )(inputs, W)

    per = NW * EC
    C = (E + per - 1) // per
    E_pad = C * per
    pad = E_pad - E
    src = jnp.concatenate(
        [edge_index[0], jnp.zeros((pad,), jnp.int32)]).reshape(NW, C, EC)
    dst = jnp.concatenate(
        [edge_index[1], jnp.zeros((pad,), jnp.int32)]).reshape(NW, C, EC)
    wts = jnp.concatenate(
        [edge_weight, jnp.zeros((pad,), jnp.float32)]).reshape(NW, C, EC)

    stripe = NS * EC
    N_pad = ((N + stripe - 1) // stripe) * stripe

    p0, p1 = _make_sc_agg(N, N_pad, D, C)(h, src, dst, wts)

    out = pl.pallas_call(
        _combine_body,
        grid=(N // BM,),
        in_specs=[pl.BlockSpec((BM, D), lambda i: (i, 0)),
                  pl.BlockSpec((BM, D), lambda i: (i, 0))],
        out_specs=pl.BlockSpec((BM, D), lambda i: (i, 0)),
        out_shape=jax.ShapeDtypeStruct((N, D), jnp.float32),
    )(p0, p1)
    return out


# speedup vs baseline: 6.5901x; 1.3571x over previous
import functools

import jax
import jax.numpy as jnp
from jax import lax
from jax.experimental import pallas as pl
from jax.experimental.pallas import tpu as pltpu
from jax.experimental.pallas import tpu_sc as plsc

NC = 2
NS = 16
NW = NC * NS
GC = 96
LANES = 16


def _mm_tanh_body(x_ref, w_ref, o_ref):
    o_ref[...] = jnp.tanh(
        lax.dot_general(x_ref[...], w_ref[...], (((1,), (0,)), ((), ())),
                        precision=lax.Precision.HIGHEST,
                        preferred_element_type=jnp.float32))


def _combine_body(a_ref, b_ref, o_ref):
    o_ref[...] = a_ref[...] + b_ref[...]


def _make_sc_agg(N, N_pad, D, C):
    mesh = plsc.VectorSubcoreMesh(core_axis_name="c", subcore_axis_name="s",
                                  num_cores=NC, num_subcores=NS)
    rows_per_tile = N_pad // NS
    assert C % 3 == 0 and rows_per_tile % GC == 0

    @functools.partial(
        pl.kernel,
        out_type=(jax.ShapeDtypeStruct((N_pad, D), jnp.float32),
                  jax.ShapeDtypeStruct((N_pad, D), jnp.float32)),
        mesh=mesh,
        compiler_params=pltpu.CompilerParams(needs_layout_passes=False),
        scratch_types=[
            pltpu.VMEM((9, 128), jnp.int32),
            pltpu.VMEM((GC, D), jnp.float32),
            pltpu.VMEM((GC, D), jnp.float32),
            pltpu.VMEM((GC, D), jnp.float32),
            pltpu.VMEM_SHARED((N_pad, D), jnp.float32),
            pltpu.SemaphoreType.DMA,
            pltpu.SemaphoreType.DMA,
            pltpu.SemaphoreType.DMA,
            pltpu.SemaphoreType.DMA,
            pltpu.SemaphoreType.DMA,
            pltpu.SemaphoreType.DMA,
            pltpu.SemaphoreType.DMA,
            pltpu.SemaphoreType.DMA,
            pltpu.SemaphoreType.DMA,
        ],
    )
    def sc_agg(h_hbm, ed_hbm, p0_hbm, p1_hbm,
               ring, b0, b1, b2, acc_sh,
               es0, es1, es2, gs0, gs1, gs2, ss0, ss1, ss2):
        cid = lax.axis_index("c")
        sid = lax.axis_index("s")
        wid = sid * NC + cid
        bufs = (b0, b1, b2)
        ess = (es0, es1, es2)
        gss = (gs0, gs1, gs2)
        sss = (ss0, ss1, ss2)
        Cm1 = C - 1

        def _zrow(r, carry):
            for j in range(D // LANES):
                b0[r, pl.ds(j * LANES, LANES)] = jnp.zeros((LANES,),
                                                           jnp.float32)
            return carry
        lax.fori_loop(0, GC, _zrow, 0)
        base = sid * rows_per_tile
        for k in range(rows_per_tile // GC):
            pltpu.sync_copy(b0, acc_sh.at[pl.ds(base + k * GC, GC)])
        plsc.subcore_barrier()

        def _refill(row, sl, sem):
            pltpu.async_copy(ed_hbm.at[wid, row],
                             ring.at[pl.ds(3 * sl, 3)], sem)

        def _ewait(sem):
            pltpu.make_async_copy(ed_hbm.at[wid, 0],
                                  ring.at[pl.ds(0, 3)], sem).wait()

        def _gather(sl, buf, sem):
            pltpu.async_copy(h_hbm.at[ring.at[3 * sl, pl.ds(0, GC)]],
                             buf, sem)

        def _gwait(buf, sem):
            pltpu.make_async_copy(h_hbm.at[ring.at[0, pl.ds(0, GC)]],
                                  buf, sem).wait()

        zidx = jnp.zeros((LANES,), jnp.int32)

        def _scatter(buf, sl, sem):
            for k in range(GC // LANES):
                dv = ring[3 * sl + 1, pl.ds(k * LANES, LANES)]
                pltpu.async_copy(buf.at[pl.ds(k * LANES, LANES)],
                                 acc_sh.at[dv], sem, add=True)

        def _sdrain(buf, sem):
            for k in range(GC // LANES):
                pltpu.make_async_copy(buf.at[pl.ds(k * LANES, LANES)],
                                      acc_sh.at[zidx], sem).wait()

        def _scale(buf, sl):
            wr = jnp.full((LANES,), 3 * sl + 2, jnp.int32)

            def _edge4(q, ecarry):
                for u in range(4):
                    e = 4 * q + u
                    wbits = plsc.load_gather(
                        ring, [wr, jnp.full((LANES,), e, jnp.int32)])
                    ws = plsc.bitcast(wbits, jnp.float32)
                    for j in range(D // LANES):
                        fs = pl.ds(j * LANES, LANES)
                        buf[e, fs] = buf[e, fs] * ws
                return ecarry
            lax.fori_loop(0, GC // 4, _edge4, 0)

        _refill(0, 0, es0)
        _refill(1, 1, es1)
        _refill(2, 2, es2)
        _ewait(es0)
        _gather(0, b0, gs0)

        def _body(i, carry):
            for k in range(3):
                r = 3 * i + k
                kp1 = (k + 1) % 3

                @pl.when(r >= 2)
                def _():
                    _sdrain(bufs[kp1], sss[kp1])
                _ewait(ess[kp1])
                _gather(kp1, bufs[kp1], gss[kp1])
                _gwait(bufs[k], gss[k])
                _scale(bufs[k], k)
                _scatter(bufs[k], k, sss[k])
                _refill(jnp.minimum(r + 3, Cm1), k, ess[k])
            return carry
        lax.fori_loop(0, C // 3, _body, 0)

        _sdrain(bufs[(C - 2) % 3], sss[(C - 2) % 3])
        _sdrain(bufs[(C - 1) % 3], sss[(C - 1) % 3])
        _gwait(bufs[C % 3], gss[C % 3])
        _ewait(ess[(C + 1) % 3])
        _ewait(ess[(C + 2) % 3])
        plsc.subcore_barrier()

        @pl.when(cid == 0)
        def _():
            pltpu.sync_copy(acc_sh.at[pl.ds(base, rows_per_tile)],
                            p0_hbm.at[pl.ds(base, rows_per_tile)])

        @pl.when(cid == 1)
        def _():
            pltpu.sync_copy(acc_sh.at[pl.ds(base, rows_per_tile)],
                            p1_hbm.at[pl.ds(base, rows_per_tile)])

    return sc_agg


def kernel(inputs, edge_index, edge_weight, W, b):
    N, D = inputs.shape
    E = edge_weight.shape[0]

    BM = 2000
    h = pl.pallas_call(
        _mm_tanh_body,
        grid=(N // BM,),
        in_specs=[pl.BlockSpec((BM, D), lambda i: (i, 0)),
                  pl.BlockSpec((D, D), lambda i: (0, 0))],
        out_specs=pl.BlockSpec((BM, D), lambda i: (i, 0)),
        out_shape=jax.ShapeDtypeStruct((N, D), jnp.float32),
    )(inputs, W)

    per = NW * GC
    C = (E + per - 1) // per
    C = ((C + 2) // 3) * 3
    E_pad = C * per
    pad = E_pad - E
    rpad = ((0, 0), (0, 0), (0, 128 - GC))
    src = jnp.pad(jnp.concatenate(
        [edge_index[0], jnp.zeros((pad,), jnp.int32)]).reshape(NW, C, GC),
        rpad)
    dst = jnp.pad(jnp.concatenate(
        [edge_index[1], jnp.zeros((pad,), jnp.int32)]).reshape(NW, C, GC),
        rpad)
    wbits = jnp.pad(jnp.concatenate(
        [lax.bitcast_convert_type(edge_weight, jnp.int32),
         jnp.zeros((pad,), jnp.int32)]).reshape(NW, C, GC), rpad)
    edata = jnp.stack([src, dst, wbits], axis=2)

    stripe = NS * GC
    N_pad = ((N + stripe - 1) // stripe) * stripe

    p0, p1 = _make_sc_agg(N, N_pad, D, C)(h, edata)

    out = pl.pallas_call(
        _combine_body,
        grid=(N // BM,),
        in_specs=[pl.BlockSpec((BM, D), lambda i: (i, 0)),
                  pl.BlockSpec((BM, D), lambda i: (i, 0))],
        out_specs=pl.BlockSpec((BM, D), lambda i: (i, 0)),
        out_shape=jax.ShapeDtypeStruct((N, D), jnp.float32),
    )(p0, p1)
    return out


# speedup vs baseline: 6.7740x; 1.0279x over previous
import functools

import jax
import jax.numpy as jnp
from jax import lax
from jax.experimental import pallas as pl
from jax.experimental.pallas import tpu as pltpu
from jax.experimental.pallas import tpu_sc as plsc

NC = 2
NS = 16
NW = NC * NS
GC = 96
LANES = 16


def _mm_tanh_body(x_ref, w_ref, o_ref):
    o_ref[...] = jnp.tanh(
        lax.dot_general(x_ref[...], w_ref[...], (((1,), (0,)), ((), ())),
                        precision=lax.Precision.HIGHEST,
                        preferred_element_type=jnp.float32))


def _combine_body(a_ref, b_ref, o_ref):
    o_ref[...] = a_ref[...] + b_ref[...]


def _make_sc_agg(N, N_pad, D, C):
    mesh = plsc.VectorSubcoreMesh(core_axis_name="c", subcore_axis_name="s",
                                  num_cores=NC, num_subcores=NS)
    rows_per_tile = N_pad // NS
    assert C % 3 == 0 and rows_per_tile % GC == 0

    @functools.partial(
        pl.kernel,
        out_type=(jax.ShapeDtypeStruct((N_pad, D), jnp.float32),
                  jax.ShapeDtypeStruct((N_pad, D), jnp.float32)),
        mesh=mesh,
        compiler_params=pltpu.CompilerParams(needs_layout_passes=False),
        scratch_types=[
            pltpu.VMEM((9, 128), jnp.int32),
            pltpu.VMEM((GC, D), jnp.float32),
            pltpu.VMEM((GC, D), jnp.float32),
            pltpu.VMEM((GC, D), jnp.float32),
            pltpu.VMEM_SHARED((N_pad, D), jnp.float32),
            pltpu.SemaphoreType.DMA,
            pltpu.SemaphoreType.DMA,
            pltpu.SemaphoreType.DMA,
            pltpu.SemaphoreType.DMA,
            pltpu.SemaphoreType.DMA,
            pltpu.SemaphoreType.DMA,
            pltpu.SemaphoreType.DMA,
            pltpu.SemaphoreType.DMA,
            pltpu.SemaphoreType.DMA,
        ],
    )
    def sc_agg(h_hbm, ed_hbm, p0_hbm, p1_hbm,
               ring, b0, b1, b2, acc_sh,
               es0, es1, es2, gs0, gs1, gs2, ss0, ss1, ss2):
        cid = lax.axis_index("c")
        sid = lax.axis_index("s")
        wid = sid * NC + cid
        bufs = (b0, b1, b2)
        ess = (es0, es1, es2)
        gss = (gs0, gs1, gs2)
        sss = (ss0, ss1, ss2)
        Cm1 = C - 1

        def _zrow(r, carry):
            for j in range(D // LANES):
                b0[r, pl.ds(j * LANES, LANES)] = jnp.zeros((LANES,),
                                                           jnp.float32)
            return carry
        lax.fori_loop(0, GC, _zrow, 0)
        base = sid * rows_per_tile
        for k in range(rows_per_tile // GC):
            pltpu.sync_copy(b0, acc_sh.at[pl.ds(base + k * GC, GC)])
        plsc.subcore_barrier()

        def _refill(row, sl, sem):
            pltpu.async_copy(ed_hbm.at[wid, row],
                             ring.at[pl.ds(3 * sl, 3)], sem)

        def _ewait(sem):
            pltpu.make_async_copy(ed_hbm.at[wid, 0],
                                  ring.at[pl.ds(0, 3)], sem).wait()

        def _gather(sl, buf, sem):
            pltpu.async_copy(h_hbm.at[ring.at[3 * sl, pl.ds(0, GC)]],
                             buf, sem)

        def _gwait(buf, sem):
            pltpu.make_async_copy(h_hbm.at[ring.at[0, pl.ds(0, GC)]],
                                  buf, sem).wait()

        zidx = jnp.zeros((LANES,), jnp.int32)

        def _scatter(buf, sl, sem):
            for k in range(GC // LANES):
                dv = ring[3 * sl + 1, pl.ds(k * LANES, LANES)]
                pltpu.async_copy(buf.at[pl.ds(k * LANES, LANES)],
                                 acc_sh.at[dv], sem, add=True)

        def _sdrain(buf, sem):
            for k in range(GC // LANES):
                pltpu.make_async_copy(buf.at[pl.ds(k * LANES, LANES)],
                                      acc_sh.at[zidx], sem).wait()

        def _scale(buf, sl):
            wr = jnp.full((LANES,), 3 * sl + 2, jnp.int32)

            @plsc.parallel_loop(0, GC, 1, unroll=4)
            def _edge(e):
                wbits = plsc.load_gather(
                    ring, [wr, jnp.full((LANES,), e, jnp.int32)])
                ws = plsc.bitcast(wbits, jnp.float32)
                for j in range(D // LANES):
                    fs = pl.ds(j * LANES, LANES)
                    buf[e, fs] = buf[e, fs] * ws

        _refill(0, 0, es0)
        _refill(1, 1, es1)
        _refill(2, 2, es2)
        _ewait(es0)
        _gather(0, b0, gs0)

        def _body(i, carry):
            for k in range(3):
                r = 3 * i + k
                kp1 = (k + 1) % 3

                @pl.when(r >= 2)
                def _():
                    _sdrain(bufs[kp1], sss[kp1])
                _ewait(ess[kp1])
                _gather(kp1, bufs[kp1], gss[kp1])
                _gwait(bufs[k], gss[k])
                _scale(bufs[k], k)
                _scatter(bufs[k], k, sss[k])
                _refill(jnp.minimum(r + 3, Cm1), k, ess[k])
            return carry
        lax.fori_loop(0, C // 3, _body, 0)

        _sdrain(bufs[(C - 2) % 3], sss[(C - 2) % 3])
        _sdrain(bufs[(C - 1) % 3], sss[(C - 1) % 3])
        _gwait(bufs[C % 3], gss[C % 3])
        _ewait(ess[(C + 1) % 3])
        _ewait(ess[(C + 2) % 3])
        plsc.subcore_barrier()

        @pl.when(cid == 0)
        def _():
            pltpu.sync_copy(acc_sh.at[pl.ds(base, rows_per_tile)],
                            p0_hbm.at[pl.ds(base, rows_per_tile)])

        @pl.when(cid == 1)
        def _():
            pltpu.sync_copy(acc_sh.at[pl.ds(base, rows_per_tile)],
                            p1_hbm.at[pl.ds(base, rows_per_tile)])

    return sc_agg


def kernel(inputs, edge_index, edge_weight, W, b):
    N, D = inputs.shape
    E = edge_weight.shape[0]

    BM = 2000
    h = pl.pallas_call(
        _mm_tanh_body,
        grid=(N // BM,),
        in_specs=[pl.BlockSpec((BM, D), lambda i: (i, 0)),
                  pl.BlockSpec((D, D), lambda i: (0, 0))],
        out_specs=pl.BlockSpec((BM, D), lambda i: (i, 0)),
        out_shape=jax.ShapeDtypeStruct((N, D), jnp.float32),
    )(inputs, W)

    per = NW * GC
    C = (E + per - 1) // per
    C = ((C + 2) // 3) * 3
    E_pad = C * per
    pad = E_pad - E
    rpad = ((0, 0), (0, 0), (0, 128 - GC))
    src = jnp.pad(jnp.concatenate(
        [edge_index[0], jnp.zeros((pad,), jnp.int32)]).reshape(NW, C, GC),
        rpad)
    dst = jnp.pad(jnp.concatenate(
        [edge_index[1], jnp.zeros((pad,), jnp.int32)]).reshape(NW, C, GC),
        rpad)
    wbits = jnp.pad(jnp.concatenate(
        [lax.bitcast_convert_type(edge_weight, jnp.int32),
         jnp.zeros((pad,), jnp.int32)]).reshape(NW, C, GC), rpad)
    edata = jnp.stack([src, dst, wbits], axis=2)

    stripe = NS * GC
    N_pad = ((N + stripe - 1) // stripe) * stripe

    p0, p1 = _make_sc_agg(N, N_pad, D, C)(h, edata)

    out = pl.pallas_call(
        _combine_body,
        grid=(N // BM,),
        in_specs=[pl.BlockSpec((BM, D), lambda i: (i, 0)),
                  pl.BlockSpec((BM, D), lambda i: (i, 0))],
        out_specs=pl.BlockSpec((BM, D), lambda i: (i, 0)),
        out_shape=jax.ShapeDtypeStruct((N, D), jnp.float32),
    )(p0, p1)
    return out
